# baseline (speedup 1.0000x reference)
import functools

import jax
import jax.numpy as jnp
from jax import lax
from jax.experimental import pallas as pl
from jax.experimental.pallas import tpu as pltpu
from jax.experimental.pallas import tpu_sc as plsc

B = 16384
L = 200
D = 64
NC, NS, LANES = 2, 16, 16
NW = NC * NS
ROWS_PER_W = B // NW
G = 4
CHUNKS = ROWS_PER_W // G
TOK = G * L
IDXW = 100
NGATH = TOK // IDXW
X2W = 100
KV = D // LANES
INV_L = 1.0 / L


def _issue_gathers(table_ref, idx_ref, rows_ref, sem):
    for j in range(NGATH):
        pltpu.async_copy(
            table_ref.at[idx_ref.at[j]],
            rows_ref.at[pl.ds(j * IDXW, IDXW)],
            sem,
        )


def _drain(table_ref, rows_ref, sem):
    pltpu.make_async_copy(table_ref.at[pl.ds(0, TOK)], rows_ref, sem).wait()


def _accumulate(rows_ref, out_stage, slot):
    for g in range(G):
        base = g * L
        zero = jnp.zeros((LANES,), jnp.float32)

        def body(i, accs, base=base):
            a = list(accs)
            for u in range(4):
                r = base + i * 4 + u
                for k in range(KV):
                    a[k] = a[k] + rows_ref[r, pl.ds(k * LANES, LANES)]
            return tuple(a)

        accs = lax.fori_loop(0, L // 4, body, (zero,) * KV, unroll=5)
        for k in range(KV):
            m = jnp.maximum(accs[k] * INV_L, 0.0)
            out_stage[slot * G + g, pl.ds(k * LANES, LANES)] = m


def _sc_bow(x2, table):
    mesh = plsc.VectorSubcoreMesh(
        core_axis_name="c", subcore_axis_name="s",
        num_cores=NC, num_subcores=NS)

    @functools.partial(
        pl.kernel,
        out_type=jax.ShapeDtypeStruct((B, D), jnp.float32),
        mesh=mesh,
        compiler_params=pltpu.CompilerParams(use_tc_tiling_on_sc=False),
        scratch_types=[
            pltpu.VMEM((NGATH, IDXW), jnp.int32),
            pltpu.VMEM((NGATH, IDXW), jnp.int32),
            pltpu.VMEM((TOK, D), jnp.float32),
            pltpu.VMEM((TOK, D), jnp.float32),
            pltpu.VMEM((2 * G, D), jnp.float32),
            pltpu.SemaphoreType.DMA,
            pltpu.SemaphoreType.DMA,
        ],
    )
    def bow(x2_ref, table_ref, out_ref,
            idx0, idx1, rows0, rows1, out_stage, sem0, sem1):
        wid = lax.axis_index("s") * NC + lax.axis_index("c")
        xrow0 = wid * (CHUNKS * NGATH)
        orow0 = wid * ROWS_PER_W

        pltpu.sync_copy(x2_ref.at[pl.ds(xrow0, NGATH)], idx0)
        _issue_gathers(table_ref, idx0, rows0, sem0)

        def step(t, carry):
            pltpu.sync_copy(
                x2_ref.at[pl.ds(xrow0 + (2 * t + 1) * NGATH, NGATH)], idx1)
            _issue_gathers(table_ref, idx1, rows1, sem1)
            _drain(table_ref, rows0, sem0)
            _accumulate(rows0, out_stage, 0)

            @pl.when(t < CHUNKS // 2 - 1)
            def _():
                pltpu.sync_copy(
                    x2_ref.at[pl.ds(xrow0 + (2 * t + 2) * NGATH, NGATH)], idx0)
                _issue_gathers(table_ref, idx0, rows0, sem0)

            _drain(table_ref, rows1, sem1)
            _accumulate(rows1, out_stage, 1)

            pltpu.sync_copy(out_stage,
                            out_ref.at[pl.ds(orow0 + t * (2 * G), 2 * G)])
            return carry

        lax.fori_loop(0, CHUNKS // 2, step, 0)

    return bow(x2, table)


def _tc_classify(m, wt, b2):
    def body(m_ref, w_ref, b_ref, o_ref):
        o_ref[...] = (
            jnp.dot(m_ref[...], w_ref[...], preferred_element_type=jnp.float32)
            + b_ref[...])

    grid = 16
    bm = B // grid
    return pl.pallas_call(
        body,
        grid=(grid,),
        in_specs=[
            pl.BlockSpec((bm, D), lambda i: (i, 0)),
            pl.BlockSpec((D, 8), lambda i: (0, 0)),
            pl.BlockSpec((1, 8), lambda i: (0, 0)),
        ],
        out_specs=pl.BlockSpec((bm, 8), lambda i: (i, 0)),
        out_shape=jax.ShapeDtypeStruct((B, 8), jnp.float32),
    )(m, wt, b2)


def kernel(x, emb_table, W, b):
    x2 = x.astype(jnp.int32).reshape(-1, X2W)
    pooled = _sc_bow(x2, emb_table)
    nc = W.shape[0]
    wt = jnp.zeros((D, 8), jnp.float32).at[:, :nc].set(W.T)
    b2 = jnp.zeros((1, 8), jnp.float32).at[0, :nc].set(b)
    logits = _tc_classify(pooled, wt, b2)
    return logits[:, :nc]


# speedup vs baseline: 1.0467x; 1.0467x over previous
import functools

import jax
import jax.numpy as jnp
from jax import lax
from jax.experimental import pallas as pl
from jax.experimental.pallas import tpu as pltpu
from jax.experimental.pallas import tpu_sc as plsc

B = 16384
L = 200
D = 64
NC, NS, LANES = 2, 16, 16
NW = NC * NS
ROWS_PER_W = B // NW
G = 4
CHUNKS = ROWS_PER_W // G
TOK = G * L
IDXW = 100
NGATH = TOK // IDXW
X2W = 100
KV = D // LANES
INV_L = 1.0 / L


def _issue_gathers(table_ref, idx_ref, rows_ref, sem):
    for j in range(NGATH):
        pltpu.async_copy(
            table_ref.at[idx_ref.at[j]],
            rows_ref.at[pl.ds(j * IDXW, IDXW)],
            sem,
        )


def _drain(table_ref, rows_ref, sem):
    pltpu.make_async_copy(table_ref.at[pl.ds(0, TOK)], rows_ref, sem).wait()


def _accumulate(rows_ref, out_stage, slot):
    for g in range(G):
        base = g * L
        zero = jnp.zeros((LANES,), jnp.float32)

        def body(i, accs, base=base):
            a = list(accs)
            for u in range(4):
                r = base + i * 4 + u
                for k in range(KV):
                    a[k] = a[k] + rows_ref[r, pl.ds(k * LANES, LANES)]
            return tuple(a)

        accs = lax.fori_loop(0, L // 4, body, (zero,) * KV, unroll=2)
        for k in range(KV):
            m = jnp.maximum(accs[k] * INV_L, 0.0)
            out_stage[slot * G + g, pl.ds(k * LANES, LANES)] = m


def _sc_bow(x2, table):
    mesh = plsc.VectorSubcoreMesh(
        core_axis_name="c", subcore_axis_name="s",
        num_cores=NC, num_subcores=NS)

    @functools.partial(
        pl.kernel,
        out_type=jax.ShapeDtypeStruct((B, D), jnp.float32),
        mesh=mesh,
        compiler_params=pltpu.CompilerParams(use_tc_tiling_on_sc=False),
        scratch_types=[
            pltpu.VMEM((NGATH, IDXW), jnp.int32),
            pltpu.VMEM((NGATH, IDXW), jnp.int32),
            pltpu.VMEM((TOK, D), jnp.float32),
            pltpu.VMEM((TOK, D), jnp.float32),
            pltpu.VMEM((2 * G, D), jnp.float32),
            pltpu.SemaphoreType.DMA,
            pltpu.SemaphoreType.DMA,
            pltpu.SemaphoreType.DMA,
            pltpu.SemaphoreType.DMA,
        ],
    )
    def bow(x2_ref, table_ref, out_ref,
            idx0, idx1, rows0, rows1, out_stage, sem0, sem1, isem0, isem1):
        wid = lax.axis_index("s") * NC + lax.axis_index("c")
        xrow0 = wid * (CHUNKS * NGATH)
        orow0 = wid * ROWS_PER_W

        def idx_start(c, idxb, isem):
            pltpu.async_copy(
                x2_ref.at[pl.ds(xrow0 + c * NGATH, NGATH)], idxb, isem)

        def idx_wait(idxb, isem):
            pltpu.make_async_copy(
                x2_ref.at[pl.ds(0, NGATH)], idxb, isem).wait()

        pltpu.sync_copy(x2_ref.at[pl.ds(xrow0, NGATH)], idx0)
        _issue_gathers(table_ref, idx0, rows0, sem0)
        idx_start(1, idx1, isem1)

        def step(t, carry):
            idx_wait(idx1, isem1)
            _issue_gathers(table_ref, idx1, rows1, sem1)
            _drain(table_ref, rows0, sem0)

            @pl.when(t < CHUNKS // 2 - 1)
            def _():
                idx_start(2 * t + 2, idx0, isem0)

            _accumulate(rows0, out_stage, 0)

            @pl.when(t < CHUNKS // 2 - 1)
            def _():
                idx_wait(idx0, isem0)
                _issue_gathers(table_ref, idx0, rows0, sem0)

            _drain(table_ref, rows1, sem1)

            @pl.when(t < CHUNKS // 2 - 1)
            def _():
                idx_start(2 * t + 3, idx1, isem1)

            _accumulate(rows1, out_stage, 1)

            pltpu.sync_copy(out_stage,
                            out_ref.at[pl.ds(orow0 + t * (2 * G), 2 * G)])
            return carry

        lax.fori_loop(0, CHUNKS // 2, step, 0)

    return bow(x2, table)


def _tc_classify(m, wt, b2):
    def body(m_ref, w_ref, b_ref, o_ref):
        o_ref[...] = (
            jnp.dot(m_ref[...], w_ref[...], preferred_element_type=jnp.float32)
            + b_ref[...])

    grid = 16
    bm = B // grid
    return pl.pallas_call(
        body,
        grid=(grid,),
        in_specs=[
            pl.BlockSpec((bm, D), lambda i: (i, 0)),
            pl.BlockSpec((D, 8), lambda i: (0, 0)),
            pl.BlockSpec((1, 8), lambda i: (0, 0)),
        ],
        out_specs=pl.BlockSpec((bm, 8), lambda i: (i, 0)),
        out_shape=jax.ShapeDtypeStruct((B, 8), jnp.float32),
    )(m, wt, b2)


def kernel(x, emb_table, W, b):
    x2 = x.astype(jnp.int32).reshape(-1, X2W)
    pooled = _sc_bow(x2, emb_table)
    nc = W.shape[0]
    wt = jnp.zeros((D, 8), jnp.float32).at[:, :nc].set(W.T)
    b2 = jnp.zeros((1, 8), jnp.float32).at[0, :nc].set(b)
    logits = _tc_classify(pooled, wt, b2)
    return logits[:, :nc]


# speedup vs baseline: 1.0517x; 1.0048x over previous
import functools

import jax
import jax.numpy as jnp
from jax import lax
from jax.experimental import pallas as pl
from jax.experimental.pallas import tpu as pltpu
from jax.experimental.pallas import tpu_sc as plsc

B = 16384
L = 200
D = 64
NC, NS, LANES = 2, 16, 16
NW = NC * NS
ROWS_PER_W = B // NW
G = 4
CHUNKS = ROWS_PER_W // G
TOK = G * L
IDXW = 100
NGATH = TOK // IDXW
X2W = 100
KV = D // LANES
INV_L = 1.0 / L


def _issue_gathers(table_ref, idx_ref, rows_ref, sem):
    for j in range(NGATH):
        pltpu.async_copy(
            table_ref.at[idx_ref.at[j]],
            rows_ref.at[pl.ds(j * IDXW, IDXW)],
            sem,
        )


def _drain(table_ref, rows_ref, sem):
    pltpu.make_async_copy(table_ref.at[pl.ds(0, TOK)], rows_ref, sem).wait()


def _accumulate(rows_ref, out_stage, slot):
    for g in range(G):
        base = g * L
        zero = jnp.zeros((LANES,), jnp.float32)

        def body(i, accs, base=base):
            a = list(accs)
            for u in range(4):
                r = base + i * 4 + u
                for k in range(KV):
                    a[k] = a[k] + rows_ref[r, pl.ds(k * LANES, LANES)]
            return tuple(a)

        accs = lax.fori_loop(0, L // 4, body, (zero,) * KV, unroll=2)
        for k in range(KV):
            m = jnp.maximum(accs[k] * INV_L, 0.0)
            out_stage[slot * G + g, pl.ds(k * LANES, LANES)] = m


def _sc_bow(x2, table):
    mesh = plsc.VectorSubcoreMesh(
        core_axis_name="c", subcore_axis_name="s",
        num_cores=NC, num_subcores=NS)

    @functools.partial(
        pl.kernel,
        out_type=jax.ShapeDtypeStruct((B, D), jnp.float32),
        mesh=mesh,
        compiler_params=pltpu.CompilerParams(use_tc_tiling_on_sc=False),
        scratch_types=[
            pltpu.VMEM((NGATH, IDXW), jnp.int32),
            pltpu.VMEM((NGATH, IDXW), jnp.int32),
            pltpu.VMEM((TOK, D), jnp.float32),
            pltpu.VMEM((TOK, D), jnp.float32),
            pltpu.VMEM((2 * G, D), jnp.float32),
            pltpu.SemaphoreType.DMA,
            pltpu.SemaphoreType.DMA,
            pltpu.SemaphoreType.DMA,
            pltpu.SemaphoreType.DMA,
            pltpu.SemaphoreType.DMA,
        ],
    )
    def bow(x2_ref, table_ref, out_ref,
            idx0, idx1, rows0, rows1, out_stage,
            sem0, sem1, isem0, isem1, osem):
        wid = lax.axis_index("s") * NC + lax.axis_index("c")
        xrow0 = wid * (CHUNKS * NGATH)
        orow0 = wid * ROWS_PER_W

        def idx_start(c, idxb, isem):
            pltpu.async_copy(
                x2_ref.at[pl.ds(xrow0 + c * NGATH, NGATH)], idxb, isem)

        def idx_wait(idxb, isem):
            pltpu.make_async_copy(
                x2_ref.at[pl.ds(0, NGATH)], idxb, isem).wait()

        pltpu.sync_copy(x2_ref.at[pl.ds(xrow0, NGATH)], idx0)
        _issue_gathers(table_ref, idx0, rows0, sem0)
        idx_start(1, idx1, isem1)

        def step(t, carry):
            idx_wait(idx1, isem1)
            _issue_gathers(table_ref, idx1, rows1, sem1)
            _drain(table_ref, rows0, sem0)

            @pl.when(t < CHUNKS // 2 - 1)
            def _():
                idx_start(2 * t + 2, idx0, isem0)

            @pl.when(t > 0)
            def _():
                pltpu.make_async_copy(
                    out_stage, out_ref.at[pl.ds(0, 2 * G)], osem).wait()

            _accumulate(rows0, out_stage, 0)

            @pl.when(t < CHUNKS // 2 - 1)
            def _():
                idx_wait(idx0, isem0)
                _issue_gathers(table_ref, idx0, rows0, sem0)

            _drain(table_ref, rows1, sem1)

            @pl.when(t < CHUNKS // 2 - 1)
            def _():
                idx_start(2 * t + 3, idx1, isem1)

            _accumulate(rows1, out_stage, 1)

            pltpu.async_copy(out_stage,
                             out_ref.at[pl.ds(orow0 + t * (2 * G), 2 * G)],
                             osem)
            return carry

        lax.fori_loop(0, CHUNKS // 2, step, 0)
        pltpu.make_async_copy(
            out_stage, out_ref.at[pl.ds(0, 2 * G)], osem).wait()

    return bow(x2, table)


def _tc_classify(m, wt, b2):
    def body(m_ref, w_ref, b_ref, o_ref):
        o_ref[...] = (
            jnp.dot(m_ref[...], w_ref[...], preferred_element_type=jnp.float32)
            + b_ref[...])

    grid = 16
    bm = B // grid
    return pl.pallas_call(
        body,
        grid=(grid,),
        in_specs=[
            pl.BlockSpec((bm, D), lambda i: (i, 0)),
            pl.BlockSpec((D, 8), lambda i: (0, 0)),
            pl.BlockSpec((1, 8), lambda i: (0, 0)),
        ],
        out_specs=pl.BlockSpec((bm, 8), lambda i: (i, 0)),
        out_shape=jax.ShapeDtypeStruct((B, 8), jnp.float32),
    )(m, wt, b2)


def kernel(x, emb_table, W, b):
    x2 = x.astype(jnp.int32).reshape(-1, X2W)
    pooled = _sc_bow(x2, emb_table)
    nc = W.shape[0]
    wt = jnp.zeros((D, 8), jnp.float32).at[:, :nc].set(W.T)
    b2 = jnp.zeros((1, 8), jnp.float32).at[0, :nc].set(b)
    logits = _tc_classify(pooled, wt, b2)
    return logits[:, :nc]
